# Initial kernel scaffold; baseline (speedup 1.0000x reference)
import jax
import jax.numpy as jnp
from jax.experimental import pallas as pl


def kernel(pc1, feature1, pc1_features, cor_features, params):
    raise NotImplementedError("write your pallas kernel here")


# speedup vs baseline: 5.9562x; 5.9562x over previous
import functools
import jax
import jax.numpy as jnp
from jax import lax
from jax.experimental import pallas as pl
from jax.experimental.pallas import tpu as pltpu
from jax.experimental.pallas import tpu_sc as plsc

_RADII2 = (4.0, 16.0, 64.0, 256.0)
_NSAMPLES = (4, 8, 16, 32)
_OFFS = (0, 4, 12, 28)
_NSLOT = 64
_TILE = 128
_DEFL = jax.lax.Precision.DEFAULT


def _dot(a, b, precision=_DEFL):
    return jax.lax.dot_general(a, b, (((1,), (0,)), ((), ())),
                               preferred_element_type=jnp.float32,
                               precision=precision)


def _lane_cumsum(x):
    t, n = x.shape
    k = 1
    while k < n:
        shifted = jnp.concatenate(
            [jnp.zeros((t, k), x.dtype), x[:, :n - k]], axis=1)
        x = x + shifted
        k *= 2
    return x


def _kernel_b1(xyzT_ref, pc1_ref, idx_ref):
    n = pc1_ref.shape[2]
    xt = xyzT_ref[0]
    pc = pc1_ref[0]
    base = pl.program_id(0) * n
    sq_r = jnp.sum(xt * xt, axis=1, keepdims=True)
    sq_c = jnp.sum(pc * pc, axis=0, keepdims=True)
    dot = _dot(xt, pc)
    d = sq_r + sq_c - 2.0 * dot
    for s in range(4):
        ns = _NSAMPLES[s]
        mask = d <= _RADII2[s]
        cnt = _lane_cumsum(mask.astype(jnp.int32))
        idx1 = None
        for p in range(1, ns + 1):
            idx_p = jnp.sum((cnt < p).astype(jnp.int32), axis=1)
            if idx1 is None:
                idx1 = idx_p
            else:
                idx_p = jnp.where(idx_p >= n, idx1, idx_p)
            idx_ref[0, _OFFS[s] + p - 1, :] = idx_p + base
    for q in range(60, _NSLOT):
        idx_ref[0, q, :] = jnp.zeros_like(idx1)


def _sc_gather(table, idxf):
    s_tot, dch = idxf.shape[0], table.shape[1]
    nw = 32
    per_w = s_tot // nw
    ch = 128
    nch = per_w // ch
    mesh = plsc.VectorSubcoreMesh(core_axis_name="c", subcore_axis_name="s")

    @functools.partial(
        pl.kernel, mesh=mesh,
        out_type=jax.ShapeDtypeStruct((s_tot, dch), jnp.float32),
        scratch_types=[
            pltpu.VMEM((ch,), jnp.int32),
            pltpu.VMEM((ch, dch), jnp.float32),
            pltpu.SemaphoreType.DMA,
        ],
    )
    def k(table_hbm, idx_hbm, out_hbm, idx_v, rows_v, sem):
        wid = lax.axis_index("s") * 2 + lax.axis_index("c")
        base = wid * per_w

        def body(c, carry):
            off = base + c * ch
            pltpu.sync_copy(idx_hbm.at[pl.ds(off, ch)], idx_v)
            pltpu.async_copy(table_hbm.at[idx_v], rows_v, sem).wait()
            pltpu.sync_copy(rows_v, out_hbm.at[pl.ds(off, ch)])
            return carry

        lax.fori_loop(0, nch, body, 0)

    return k(table, idxf)


def _kernel_b2(g_ref, xyzT_ref, w1_ref, b1_ref, w2_ref, b2_ref,
               w3_ref, b3_ref, m2w_ref, m2b_ref, prop_ref):
    xt = xyzT_ref[0]
    for s in range(4):
        ns = _NSAMPLES[s]
        m = None

        def slot_mlp(xe):
            gx = xe[:, 0:3] - xt
            h = jnp.concatenate([gx, xe[:, 3:]], axis=1)
            h = jnp.maximum(_dot(h, w1_ref[s]) + b1_ref[s], 0.0)
            h = jnp.maximum(_dot(h, w2_ref[s]) + b2_ref[s], 0.0)
            return jnp.maximum(_dot(h, w3_ref[s]) + b3_ref[s], 0.0)

        for p in range(ns):
            m_p = slot_mlp(g_ref[0, _OFFS[s] + p][:, :134])
            m = m_p if m is None else jnp.maximum(m, m_p)
        for l in range(3):
            m = jnp.maximum(_dot(m, m2w_ref[s, l]) + m2b_ref[s, l], 0.0)
        prop_ref[0, :, s * 8:(s + 1) * 8] = m


def _kernel_c(prop_ref, w5_ref, b5_ref, w6_ref, b6_ref, w7_ref, b7_ref,
              w8_ref, b8_ref, out_ref):
    p = prop_ref[0]
    gmax = jnp.max(p, axis=0, keepdims=True)
    x = jnp.concatenate([p, jnp.broadcast_to(gmax, p.shape)], axis=1)
    h = jnp.maximum(_dot(x, w5_ref[...]) + b5_ref[...], 0.0)
    h = jnp.maximum(_dot(h, w6_ref[...]) + b6_ref[...], 0.0)
    h = jnp.maximum(_dot(h, w7_ref[...]) + b7_ref[...], 0.0)
    out_ref[0] = _dot(h, w8_ref[...]) + b8_ref[...]


def kernel(pc1, feature1, pc1_features, cor_features, params):
    b, _, n = pc1.shape
    t = _TILE
    nt = n // t

    xyzT = jnp.transpose(pc1, (0, 2, 1))
    embT = jnp.transpose(
        jnp.concatenate([feature1, pc1_features, cor_features], axis=1),
        (0, 2, 1))
    table = jnp.concatenate(
        [xyzT, embT, jnp.zeros((b, n, 122), jnp.float32)],
        axis=2).reshape(b * n, 256)

    sc = params["scales"]
    w1 = jnp.stack([s["mlp_w"][0].T for s in sc])
    b1 = jnp.stack([s["mlp_b"][0] for s in sc])
    w2 = jnp.stack([s["mlp_w"][1].T for s in sc])
    b2 = jnp.stack([s["mlp_b"][1] for s in sc])
    w3 = jnp.stack([s["mlp_w"][2].T for s in sc])
    b3 = jnp.stack([s["mlp_b"][2] for s in sc])
    m2w = jnp.stack([jnp.stack([s["mlp2_w"][l].T for l in range(3)])
                     for s in sc])
    m2b = jnp.stack([jnp.stack([s["mlp2_b"][l] for l in range(3)])
                     for s in sc])
    fpw = [w.T for w in params["fp_w"]]
    fpb = [bb[None, :] for bb in params["fp_b"]]

    idx = pl.pallas_call(
        _kernel_b1,
        grid=(b, nt),
        in_specs=[
            pl.BlockSpec((1, t, 3), lambda i, r: (i, r, 0)),
            pl.BlockSpec((1, 3, n), lambda i, r: (i, 0, 0)),
        ],
        out_specs=pl.BlockSpec((1, _NSLOT, t), lambda i, r: (i, 0, r)),
        out_shape=jax.ShapeDtypeStruct((b, _NSLOT, n), jnp.int32),
    )(xyzT, pc1)

    gathered = _sc_gather(table, idx.reshape(b * _NSLOT * n))
    gathered = gathered.reshape(b, _NSLOT, n, 256)

    prop = pl.pallas_call(
        _kernel_b2,
        grid=(b, nt),
        in_specs=[
            pl.BlockSpec((1, _NSLOT, t, 256), lambda i, r: (i, 0, r, 0)),
            pl.BlockSpec((1, t, 3), lambda i, r: (i, r, 0)),
            pl.BlockSpec((4, 134, 64), lambda i, r: (0, 0, 0)),
            pl.BlockSpec((4, 64), lambda i, r: (0, 0)),
            pl.BlockSpec((4, 64, 32), lambda i, r: (0, 0, 0)),
            pl.BlockSpec((4, 32), lambda i, r: (0, 0)),
            pl.BlockSpec((4, 32, 8), lambda i, r: (0, 0, 0)),
            pl.BlockSpec((4, 8), lambda i, r: (0, 0)),
            pl.BlockSpec((4, 3, 8, 8), lambda i, r: (0, 0, 0, 0)),
            pl.BlockSpec((4, 3, 8), lambda i, r: (0, 0, 0)),
        ],
        out_specs=pl.BlockSpec((1, t, 32), lambda i, r: (i, r, 0)),
        out_shape=jax.ShapeDtypeStruct((b, n, 32), jnp.float32),
    )(gathered, xyzT, w1, b1, w2, b2, w3, b3, m2w, m2b)

    out = pl.pallas_call(
        _kernel_c,
        grid=(b,),
        in_specs=[pl.BlockSpec((1, n, 32), lambda i: (i, 0, 0))] +
                 [pl.BlockSpec(w.shape, lambda i, _nd=w.ndim: (0,) * _nd)
                  for pair in zip(fpw, fpb) for w in pair],
        out_specs=pl.BlockSpec((1, n, 3), lambda i: (i, 0, 0)),
        out_shape=jax.ShapeDtypeStruct((b, n, 3), jnp.float32),
    )(prop, fpw[0], fpb[0], fpw[1], fpb[1], fpw[2], fpb[2], fpw[3], fpb[3])

    return jnp.transpose(out, (0, 2, 1))


# speedup vs baseline: 8.2915x; 1.3921x over previous
import functools
import jax
import jax.numpy as jnp
from jax import lax
from jax.experimental import pallas as pl
from jax.experimental.pallas import tpu as pltpu
from jax.experimental.pallas import tpu_sc as plsc

_RADII2 = (4.0, 16.0, 64.0, 256.0)
_NSAMPLES = (4, 8, 16, 32)
_OFFS = (0, 4, 12, 28)
_NSLOT = 64
_TILE = 128
_DEFL = jax.lax.Precision.DEFAULT


def _dot(a, b, precision=_DEFL):
    return jax.lax.dot_general(a, b, (((1,), (0,)), ((), ())),
                               preferred_element_type=jnp.float32,
                               precision=precision)


def _lane_cumsum(x):
    t, n = x.shape
    k = 1
    while k < n:
        shifted = jnp.concatenate(
            [jnp.zeros((t, k), x.dtype), x[:, :n - k]], axis=1)
        x = x + shifted
        k *= 2
    return x


def _kernel_b1(xyzT_ref, pc1_ref, idx_ref):
    n = pc1_ref.shape[2]
    xt = xyzT_ref[0]
    pc = pc1_ref[0]
    base = pl.program_id(0) * n
    sq_r = jnp.sum(xt * xt, axis=1, keepdims=True)
    sq_c = jnp.sum(pc * pc, axis=0, keepdims=True)
    dot = _dot(xt, pc)
    d = sq_r + sq_c - 2.0 * dot
    for s in range(4):
        ns = _NSAMPLES[s]
        mask = d <= _RADII2[s]
        cnt = _lane_cumsum(mask.astype(jnp.int32))
        idx1 = None
        for p in range(1, ns + 1):
            idx_p = jnp.sum((cnt < p).astype(jnp.int32), axis=1)
            if idx1 is None:
                idx1 = idx_p
            else:
                idx_p = jnp.where(idx_p >= n, idx1, idx_p)
            idx_ref[0, _OFFS[s] + p - 1, :] = idx_p + base
    for q in range(60, _NSLOT):
        idx_ref[0, q, :] = jnp.zeros_like(idx1)


def _sc_gather(table, idxf):
    s_tot, dch = idxf.shape[0], table.shape[1]
    nw = 32
    per_w = s_tot // nw
    ch = 64
    nb = 4
    nround = per_w // (ch * nb)
    mesh = plsc.VectorSubcoreMesh(core_axis_name="c", subcore_axis_name="s")

    @functools.partial(
        pl.kernel, mesh=mesh,
        out_type=jax.ShapeDtypeStruct((s_tot, dch), jnp.float32),
        scratch_types=[
            pltpu.VMEM((nb, ch), jnp.int32),
            pltpu.VMEM((nb, ch, dch), jnp.float32),
            pltpu.SemaphoreType.DMA,
            pltpu.SemaphoreType.DMA,
        ],
    )
    def k(table_hbm, idx_hbm, out_hbm, idx_v, rows_v, gsem, osem):
        wid = lax.axis_index("s") * 2 + lax.axis_index("c")
        base = wid * per_w

        def body(r, carry):
            off0 = base + r * (ch * nb)
            gh = []
            for j in range(nb):
                pltpu.sync_copy(idx_hbm.at[pl.ds(off0 + j * ch, ch)],
                                idx_v.at[j])
                gh.append(pltpu.async_copy(table_hbm.at[idx_v.at[j]],
                                           rows_v.at[j], gsem))
            oh = []
            for j in range(nb):
                gh[j].wait()
                oh.append(pltpu.async_copy(
                    rows_v.at[j], out_hbm.at[pl.ds(off0 + j * ch, ch)], osem))
            for j in range(nb):
                oh[j].wait()
            return carry

        lax.fori_loop(0, nround, body, 0)

    return k(table, idxf)


def _kernel_b2(g_ref, xyzT_ref, w1_ref, b1_ref, w2_ref, b2_ref,
               w3_ref, b3_ref, m2w_ref, m2b_ref, prop_ref):
    xt = xyzT_ref[0]
    for s in range(4):
        ns = _NSAMPLES[s]
        m = None

        def slot_mlp(xe):
            gx = xe[:, 0:3] - xt
            h = jnp.concatenate([gx, xe[:, 3:]], axis=1)
            h = jnp.maximum(_dot(h, w1_ref[s]) + b1_ref[s], 0.0)
            h = jnp.maximum(_dot(h, w2_ref[s]) + b2_ref[s], 0.0)
            return jnp.maximum(_dot(h, w3_ref[s]) + b3_ref[s], 0.0)

        for p in range(ns):
            m_p = slot_mlp(g_ref[0, _OFFS[s] + p][:, :134])
            m = m_p if m is None else jnp.maximum(m, m_p)
        for l in range(3):
            m = jnp.maximum(_dot(m, m2w_ref[s, l]) + m2b_ref[s, l], 0.0)
        prop_ref[0, :, s * 8:(s + 1) * 8] = m


def _kernel_c(prop_ref, w5_ref, b5_ref, w6_ref, b6_ref, w7_ref, b7_ref,
              w8_ref, b8_ref, out_ref):
    p = prop_ref[0]
    gmax = jnp.max(p, axis=0, keepdims=True)
    x = jnp.concatenate([p, jnp.broadcast_to(gmax, p.shape)], axis=1)
    h = jnp.maximum(_dot(x, w5_ref[...]) + b5_ref[...], 0.0)
    h = jnp.maximum(_dot(h, w6_ref[...]) + b6_ref[...], 0.0)
    h = jnp.maximum(_dot(h, w7_ref[...]) + b7_ref[...], 0.0)
    out_ref[0] = _dot(h, w8_ref[...]) + b8_ref[...]


def kernel(pc1, feature1, pc1_features, cor_features, params):
    b, _, n = pc1.shape
    t = _TILE
    nt = n // t

    xyzT = jnp.transpose(pc1, (0, 2, 1))
    embT = jnp.transpose(
        jnp.concatenate([feature1, pc1_features, cor_features], axis=1),
        (0, 2, 1))
    table = jnp.concatenate(
        [xyzT, embT, jnp.zeros((b, n, 122), jnp.float32)],
        axis=2).reshape(b * n, 256)

    sc = params["scales"]
    w1 = jnp.stack([s["mlp_w"][0].T for s in sc])
    b1 = jnp.stack([s["mlp_b"][0] for s in sc])
    w2 = jnp.stack([s["mlp_w"][1].T for s in sc])
    b2 = jnp.stack([s["mlp_b"][1] for s in sc])
    w3 = jnp.stack([s["mlp_w"][2].T for s in sc])
    b3 = jnp.stack([s["mlp_b"][2] for s in sc])
    m2w = jnp.stack([jnp.stack([s["mlp2_w"][l].T for l in range(3)])
                     for s in sc])
    m2b = jnp.stack([jnp.stack([s["mlp2_b"][l] for l in range(3)])
                     for s in sc])
    fpw = [w.T for w in params["fp_w"]]
    fpb = [bb[None, :] for bb in params["fp_b"]]

    idx = pl.pallas_call(
        _kernel_b1,
        grid=(b, nt),
        in_specs=[
            pl.BlockSpec((1, t, 3), lambda i, r: (i, r, 0)),
            pl.BlockSpec((1, 3, n), lambda i, r: (i, 0, 0)),
        ],
        out_specs=pl.BlockSpec((1, _NSLOT, t), lambda i, r: (i, 0, r)),
        out_shape=jax.ShapeDtypeStruct((b, _NSLOT, n), jnp.int32),
    )(xyzT, pc1)

    gathered = _sc_gather(table, idx[:, :60, :].reshape(b * 60 * n))
    gathered = gathered.reshape(b, 60, n, 256)

    prop = pl.pallas_call(
        _kernel_b2,
        grid=(b, nt),
        in_specs=[
            pl.BlockSpec((1, 60, t, 256), lambda i, r: (i, 0, r, 0)),
            pl.BlockSpec((1, t, 3), lambda i, r: (i, r, 0)),
            pl.BlockSpec((4, 134, 64), lambda i, r: (0, 0, 0)),
            pl.BlockSpec((4, 64), lambda i, r: (0, 0)),
            pl.BlockSpec((4, 64, 32), lambda i, r: (0, 0, 0)),
            pl.BlockSpec((4, 32), lambda i, r: (0, 0)),
            pl.BlockSpec((4, 32, 8), lambda i, r: (0, 0, 0)),
            pl.BlockSpec((4, 8), lambda i, r: (0, 0)),
            pl.BlockSpec((4, 3, 8, 8), lambda i, r: (0, 0, 0, 0)),
            pl.BlockSpec((4, 3, 8), lambda i, r: (0, 0, 0)),
        ],
        out_specs=pl.BlockSpec((1, t, 32), lambda i, r: (i, r, 0)),
        out_shape=jax.ShapeDtypeStruct((b, n, 32), jnp.float32),
    )(gathered, xyzT, w1, b1, w2, b2, w3, b3, m2w, m2b)

    out = pl.pallas_call(
        _kernel_c,
        grid=(b,),
        in_specs=[pl.BlockSpec((1, n, 32), lambda i: (i, 0, 0))] +
                 [pl.BlockSpec(w.shape, lambda i, _nd=w.ndim: (0,) * _nd)
                  for pair in zip(fpw, fpb) for w in pair],
        out_specs=pl.BlockSpec((1, n, 3), lambda i: (i, 0, 0)),
        out_shape=jax.ShapeDtypeStruct((b, n, 3), jnp.float32),
    )(prop, fpw[0], fpb[0], fpw[1], fpb[1], fpw[2], fpb[2], fpw[3], fpb[3])

    return jnp.transpose(out, (0, 2, 1))
